# baseline (speedup 1.0000x reference)
import functools

import jax
import jax.numpy as jnp
from jax import lax
from jax.experimental import pallas as pl
from jax.experimental.pallas import tpu as pltpu

B, E, V, D = 64, 100, 50, 768
G = 8
STEPS = B // G
RE, RV = G * E, G * V


def _bce_rows(z_all, bias, umask, y, n_per_batch, g):
    rows = z_all.shape[0]
    r_iota = lax.broadcasted_iota(jnp.int32, (rows, B), 0)
    c_iota = lax.broadcasted_iota(jnp.int32, (rows, B), 1)
    lo = (c_iota - g * G) * n_per_batch
    sel = (r_iota >= lo) & (r_iota < lo + n_per_batch)
    w = (z_all + bias) * umask
    bce = jnp.maximum(w, 0.0) - w * y + jnp.log1p(jnp.exp(-jnp.abs(w)))
    return jnp.sum(jnp.where(sel, bce, 0.0), axis=(0, 1), keepdims=True)


def _fused_kernel(ent_ref, ev_ref, sr_ref, emask_ref, vmask_ref,
                  elab_ref, vlab_ref, wa_ref, we_ref, ba_ref, be_ref,
                  out_ref, uat_scr, uet_scr):
    g = pl.program_id(0)

    @pl.when(g == 0)
    def _init():
        sr = sr_ref[...]
        uat_scr[...] = lax.dot_general(
            wa_ref[...], sr, (((1,), (1,)), ((), ())),
            preferred_element_type=jnp.float32).astype(jnp.bfloat16)
        uet_scr[...] = lax.dot_general(
            we_ref[...], sr, (((1,), (1,)), ((), ())),
            preferred_element_type=jnp.float32).astype(jnp.bfloat16)
        out_ref[...] = jnp.zeros((1, 1), jnp.float32)

    za = lax.dot_general(ent_ref[...].astype(jnp.bfloat16), uat_scr[...],
                         (((1,), (0,)), ((), ())),
                         preferred_element_type=jnp.float32)
    zv = lax.dot_general(ev_ref[...].astype(jnp.bfloat16), uet_scr[...],
                         (((1,), (0,)), ((), ())),
                         preferred_element_type=jnp.float32)

    sa = _bce_rows(za, ba_ref[0], emask_ref[...], elab_ref[...], E, g)
    sv = _bce_rows(zv, be_ref[0], vmask_ref[...], vlab_ref[...], V, g)

    out_ref[...] += (0.5 / (B * E)) * sa + (0.5 / (B * V)) * sv


@functools.partial(jax.jit, static_argnames=())
def kernel(entity_mat, sr_vec, ev_mat, entity_mask, evidence_mask,
           entity_labels, evidence_labels, W_answer, b_answer,
           W_evidence, b_evidence):
    ent_flat = entity_mat.reshape(B * E, D)
    ev_flat = ev_mat.reshape(B * V, D)
    emask = entity_mask.reshape(B * E, 1)
    vmask = evidence_mask.reshape(B * V, 1)
    elab = entity_labels.astype(jnp.float32).reshape(B * E, 1)
    vlab = evidence_labels.astype(jnp.float32).reshape(B * V, 1)

    out = pl.pallas_call(
        _fused_kernel,
        grid=(STEPS,),
        in_specs=[
            pl.BlockSpec((RE, D), lambda g: (g, 0)),
            pl.BlockSpec((RV, D), lambda g: (g, 0)),
            pl.BlockSpec((B, D), lambda g: (0, 0)),
            pl.BlockSpec((RE, 1), lambda g: (g, 0)),
            pl.BlockSpec((RV, 1), lambda g: (g, 0)),
            pl.BlockSpec((RE, 1), lambda g: (g, 0)),
            pl.BlockSpec((RV, 1), lambda g: (g, 0)),
            pl.BlockSpec((D, D), lambda g: (0, 0)),
            pl.BlockSpec((D, D), lambda g: (0, 0)),
            pl.BlockSpec(memory_space=pltpu.SMEM),
            pl.BlockSpec(memory_space=pltpu.SMEM),
        ],
        out_specs=pl.BlockSpec((1, 1), lambda g: (0, 0)),
        out_shape=jax.ShapeDtypeStruct((1, 1), jnp.float32),
        scratch_shapes=[
            pltpu.VMEM((D, B), jnp.bfloat16),
            pltpu.VMEM((D, B), jnp.bfloat16),
        ],
    )(ent_flat, ev_flat, sr_vec, emask, vmask, elab, vlab,
      W_answer[0], W_evidence[0], b_answer, b_evidence)
    return out[0, 0]


# speedup vs baseline: 1.5505x; 1.5505x over previous
import functools

import jax
import jax.numpy as jnp
from jax import lax
from jax.experimental import pallas as pl
from jax.experimental.pallas import tpu as pltpu

B, E, V, D = 64, 100, 50, 768
G = 8
STEPS = B // G


def _bce_group(x_ref, ut_scr, mask_ref, lab_ref, bias, n, g):
    z = lax.dot_general(x_ref[...].astype(jnp.bfloat16), ut_scr[...],
                        (((2,), (0,)), ((), ())),
                        preferred_element_type=jnp.float32)
    bg = lax.broadcasted_iota(jnp.int32, (G, n, B), 0)
    c = lax.broadcasted_iota(jnp.int32, (G, n, B), 2)
    zd = jnp.sum(jnp.where(c == g * G + bg, z, 0.0), axis=2)
    w = (zd + bias) * mask_ref[...]
    y = lab_ref[...].astype(jnp.float32)
    bce = jnp.maximum(w, 0.0) - w * y + jnp.log1p(jnp.exp(-jnp.abs(w)))
    return jnp.sum(bce, axis=(0, 1), keepdims=True)


def _fused_kernel(ent_ref, ev_ref, sr_ref, emask_ref, vmask_ref,
                  elab_ref, vlab_ref, wa_ref, we_ref, ba_ref, be_ref,
                  out_ref, uat_scr, uet_scr):
    g = pl.program_id(0)

    @pl.when(g == 0)
    def _init():
        sr = sr_ref[...]
        uat_scr[...] = lax.dot_general(
            wa_ref[0], sr, (((1,), (1,)), ((), ())),
            preferred_element_type=jnp.float32).astype(jnp.bfloat16)
        uet_scr[...] = lax.dot_general(
            we_ref[0], sr, (((1,), (1,)), ((), ())),
            preferred_element_type=jnp.float32).astype(jnp.bfloat16)
        out_ref[...] = jnp.zeros((1, 1), jnp.float32)

    sa = _bce_group(ent_ref, uat_scr, emask_ref, elab_ref, ba_ref[0], E, g)
    sv = _bce_group(ev_ref, uet_scr, vmask_ref, vlab_ref, be_ref[0], V, g)
    out_ref[...] += (0.5 / (B * E)) * sa + (0.5 / (B * V)) * sv


@functools.partial(jax.jit, static_argnames=())
def kernel(entity_mat, sr_vec, ev_mat, entity_mask, evidence_mask,
           entity_labels, evidence_labels, W_answer, b_answer,
           W_evidence, b_evidence):
    out = pl.pallas_call(
        _fused_kernel,
        grid=(STEPS,),
        in_specs=[
            pl.BlockSpec((G, E, D), lambda g: (g, 0, 0)),
            pl.BlockSpec((G, V, D), lambda g: (g, 0, 0)),
            pl.BlockSpec((B, D), lambda g: (0, 0)),
            pl.BlockSpec((G, E), lambda g: (g, 0)),
            pl.BlockSpec((G, V), lambda g: (g, 0)),
            pl.BlockSpec((G, E), lambda g: (g, 0)),
            pl.BlockSpec((G, V), lambda g: (g, 0)),
            pl.BlockSpec((1, D, D), lambda g: (0, 0, 0)),
            pl.BlockSpec((1, D, D), lambda g: (0, 0, 0)),
            pl.BlockSpec(memory_space=pltpu.SMEM),
            pl.BlockSpec(memory_space=pltpu.SMEM),
        ],
        out_specs=pl.BlockSpec((1, 1), lambda g: (0, 0)),
        out_shape=jax.ShapeDtypeStruct((1, 1), jnp.float32),
        scratch_shapes=[
            pltpu.VMEM((D, B), jnp.bfloat16),
            pltpu.VMEM((D, B), jnp.bfloat16),
        ],
    )(entity_mat, ev_mat, sr_vec, entity_mask, evidence_mask,
      entity_labels, evidence_labels, W_answer, W_evidence,
      b_answer, b_evidence)
    return out[0, 0]
